# baseline (device time: 552169 ns/iter reference)
import functools

import jax
import jax.numpy as jnp
from jax import lax
from jax.experimental import pallas as pl
from jax.experimental.pallas import tpu as pltpu

N_DEV = 32
B, Sq, D = 2, 256, 768
Hq, Dh = 8, 64
Dq = Hq * Dh
SCALE = 0.125


def _expand(a):
    return jnp.concatenate(
        [jnp.broadcast_to(a[:, h : h + 1], (Sq, Dh)) for h in range(Hq)], axis=1
    )


def kernel(x, Wq, Wo, K_ext, V_ext):
    skv = K_ext.shape[1]

    K2 = jnp.transpose(K_ext, (0, 2, 3, 1)).reshape(B * Hq, Dh, skv)
    V2 = jnp.transpose(V_ext, (0, 2, 1, 3)).reshape(B * Hq, skv, Dh)

    def body(
        x_ref,
        wq_ref,
        wo_ref,
        k_ref,
        v_ref,
        out_ref,
        comm_o,
        comm_s,
        send_sems_o,
        recv_sems_o,
        send_sems_s,
        recv_sems_s,
    ):
        my = lax.axis_index("i")
        left = (my + N_DEV - 1) % N_DEV
        right = (my + 1) % N_DEV

        barrier_sem = pltpu.get_barrier_semaphore()
        for nbr in (left, right):
            pl.semaphore_signal(
                barrier_sem,
                inc=1,
                device_id=(nbr,),
                device_id_type=pl.DeviceIdType.MESH,
            )
        pl.semaphore_wait(barrier_sem, 2)

        acc_o, acc_m, acc_l = [], [], []
        for b in range(B):
            qb = jnp.dot(x_ref[b], wq_ref[...], preferred_element_type=jnp.float32)
            o_bands, m_cols, l_cols = [], [], []
            for h in range(Hq):
                bh = b * Hq + h
                qh = qb[:, h * Dh : (h + 1) * Dh]
                s = (
                    jnp.dot(qh, k_ref[bh], preferred_element_type=jnp.float32)
                    * SCALE
                )
                mh = jnp.max(s, axis=1, keepdims=True)
                p = jnp.exp(s - mh)
                lh = jnp.sum(p, axis=1, keepdims=True)
                oh = jnp.dot(p, v_ref[bh], preferred_element_type=jnp.float32)
                o_bands.append(oh)
                m_cols.append(mh)
                l_cols.append(lh)
            acc_o.append(jnp.concatenate(o_bands, axis=1))
            acc_m.append(jnp.concatenate(m_cols, axis=1))
            acc_l.append(jnp.concatenate(l_cols, axis=1))
            comm_o[0, b] = acc_o[b]
            comm_s[0, b, :, 0:Hq] = acc_m[b]
            comm_s[0, b, :, Hq : 2 * Hq] = acc_l[b]

        for hop in range(N_DEV - 1):
            s_slot = hop % 2
            r_slot = (hop + 1) % 2
            rdma_o = pltpu.make_async_remote_copy(
                src_ref=comm_o.at[s_slot],
                dst_ref=comm_o.at[r_slot],
                send_sem=send_sems_o.at[s_slot],
                recv_sem=recv_sems_o.at[r_slot],
                device_id=(right,),
                device_id_type=pl.DeviceIdType.MESH,
            )
            rdma_s = pltpu.make_async_remote_copy(
                src_ref=comm_s.at[s_slot],
                dst_ref=comm_s.at[r_slot],
                send_sem=send_sems_s.at[s_slot],
                recv_sem=recv_sems_s.at[r_slot],
                device_id=(right,),
                device_id_type=pl.DeviceIdType.MESH,
            )
            rdma_o.start()
            rdma_s.start()
            rdma_o.wait()
            rdma_s.wait()

            for b in range(B):
                m_in = comm_s[r_slot, b, :, 0:Hq]
                l_in = comm_s[r_slot, b, :, Hq : 2 * Hq]
                o_in = comm_o[r_slot, b]
                m_new = jnp.maximum(acc_m[b], m_in)
                a_acc = jnp.exp(acc_m[b] - m_new)
                a_in = jnp.exp(m_in - m_new)
                acc_l[b] = a_acc * acc_l[b] + a_in * l_in
                acc_o[b] = _expand(a_acc) * acc_o[b] + _expand(a_in) * o_in
                acc_m[b] = m_new

        for b in range(B):
            o = acc_o[b] / _expand(acc_l[b])
            out_ref[b] = jnp.dot(o, wo_ref[...], preferred_element_type=jnp.float32)

        @functools.partial(
            pl.run_scoped, second_barrier=pltpu.SemaphoreType.REGULAR
        )
        def _(second_barrier):
            for nbr in (left, right):
                pl.semaphore_signal(
                    second_barrier,
                    inc=1,
                    device_id=(nbr,),
                    device_id_type=pl.DeviceIdType.MESH,
                )
            pl.semaphore_wait(second_barrier, 2)

    return pl.pallas_call(
        body,
        out_shape=jax.ShapeDtypeStruct((B, Sq, D), jnp.float32),
        in_specs=[pl.BlockSpec(memory_space=pltpu.VMEM)] * 5,
        out_specs=pl.BlockSpec(memory_space=pltpu.VMEM),
        scratch_shapes=[
            pltpu.VMEM((2, B, Sq, Dq), jnp.float32),
            pltpu.VMEM((2, B, Sq, 2 * Hq), jnp.float32),
            pltpu.SemaphoreType.DMA((2,)),
            pltpu.SemaphoreType.DMA((2,)),
            pltpu.SemaphoreType.DMA((2,)),
            pltpu.SemaphoreType.DMA((2,)),
        ],
        compiler_params=pltpu.CompilerParams(collective_id=0),
    )(x, Wq, Wo, K2, V2)


# device time: 427967 ns/iter; 1.2902x vs baseline; 1.2902x over previous
import functools

import jax
import jax.numpy as jnp
from jax import lax
from jax.experimental import pallas as pl
from jax.experimental.pallas import tpu as pltpu

N_DEV = 32
B, Sq, D = 2, 256, 768
Hq, Dh = 8, 64
Dq = Hq * Dh
NB = B * Hq
W_O = B * Dq
W_S = 2 * NB
CW = N_DEV // 2
CCW = N_DEV - 1 - CW
SCALE = 0.125


def _expand(a):
    return jnp.concatenate(
        [jnp.broadcast_to(a[:, k : k + 1], (Sq, Dh)) for k in range(NB)], axis=1
    )


def kernel(x, Wq, Wo, K_ext, V_ext):
    skv = K_ext.shape[1]

    K2 = jnp.transpose(K_ext, (0, 2, 3, 1)).reshape(NB, Dh, skv)
    V2 = jnp.transpose(V_ext, (0, 2, 1, 3)).reshape(NB, skv, Dh)

    def body(
        x_ref,
        wq_ref,
        wo_ref,
        k_ref,
        v_ref,
        out_ref,
        self_o,
        self_s,
        cw_o,
        cw_s,
        ccw_o,
        ccw_s,
        cw_o_ss, cw_o_rs, cw_s_ss, cw_s_rs,
        ccw_o_ss, ccw_o_rs, ccw_s_ss, ccw_s_rs,
    ):
        my = lax.axis_index("i")
        left = (my + N_DEV - 1) % N_DEV
        right = (my + 1) % N_DEV

        def cw_pair(h):
            src_o = self_o if h == 0 else cw_o.at[h - 1]
            src_s = self_s if h == 0 else cw_s.at[h - 1]
            ro = pltpu.make_async_remote_copy(
                src_ref=src_o, dst_ref=cw_o.at[h],
                send_sem=cw_o_ss.at[h], recv_sem=cw_o_rs.at[h],
                device_id=(right,), device_id_type=pl.DeviceIdType.MESH,
            )
            rs = pltpu.make_async_remote_copy(
                src_ref=src_s, dst_ref=cw_s.at[h],
                send_sem=cw_s_ss.at[h], recv_sem=cw_s_rs.at[h],
                device_id=(right,), device_id_type=pl.DeviceIdType.MESH,
            )
            return ro, rs

        def ccw_pair(h):
            src_o = self_o if h == 0 else ccw_o.at[h - 1]
            src_s = self_s if h == 0 else ccw_s.at[h - 1]
            ro = pltpu.make_async_remote_copy(
                src_ref=src_o, dst_ref=ccw_o.at[h],
                send_sem=ccw_o_ss.at[h], recv_sem=ccw_o_rs.at[h],
                device_id=(left,), device_id_type=pl.DeviceIdType.MESH,
            )
            rs = pltpu.make_async_remote_copy(
                src_ref=src_s, dst_ref=ccw_s.at[h],
                send_sem=ccw_s_ss.at[h], recv_sem=ccw_s_rs.at[h],
                device_id=(left,), device_id_type=pl.DeviceIdType.MESH,
            )
            return ro, rs

        barrier_sem = pltpu.get_barrier_semaphore()
        for nbr in (left, right):
            pl.semaphore_signal(
                barrier_sem, inc=1,
                device_id=(nbr,), device_id_type=pl.DeviceIdType.MESH,
            )
        pl.semaphore_wait(barrier_sem, 2)

        o_bands, m_cols, l_cols = [], [], []
        for b in range(B):
            qb = jnp.dot(x_ref[b], wq_ref[...], preferred_element_type=jnp.float32)
            for h in range(Hq):
                bh = b * Hq + h
                qh = qb[:, h * Dh : (h + 1) * Dh]
                s = (
                    jnp.dot(qh, k_ref[bh], preferred_element_type=jnp.float32)
                    * SCALE
                )
                mh = jnp.max(s, axis=1, keepdims=True)
                p = jnp.exp(s - mh)
                lh = jnp.sum(p, axis=1, keepdims=True)
                oh = jnp.dot(p, v_ref[bh], preferred_element_type=jnp.float32)
                o_bands.append(oh)
                m_cols.append(mh)
                l_cols.append(lh)
        acc_o = jnp.concatenate(o_bands, axis=1)
        acc_m = jnp.concatenate(m_cols, axis=1)
        acc_l = jnp.concatenate(l_cols, axis=1)
        self_o[...] = acc_o
        self_s[:, 0:NB] = acc_m
        self_s[:, NB:W_S] = acc_l

        for r in cw_pair(0) + ccw_pair(0):
            r.start()

        def combine(acc, c_o, c_s):
            o, m, l = acc
            m_in = c_s[:, 0:NB]
            l_in = c_s[:, NB:W_S]
            m_new = jnp.maximum(m, m_in)
            a_acc = jnp.exp(m - m_new)
            a_in = jnp.exp(m_in - m_new)
            l_new = a_acc * l + a_in * l_in
            o_new = _expand(a_acc) * o + _expand(a_in) * c_o
            return o_new, m_new, l_new

        acc = (acc_o, acc_m, acc_l)
        for h in range(CW):
            ro, rs = cw_pair(h)
            ro.wait_recv()
            rs.wait_recv()
            if h + 1 < CW:
                for r in cw_pair(h + 1):
                    r.start()
            if h < CCW:
                ro2, rs2 = ccw_pair(h)
                ro2.wait_recv()
                rs2.wait_recv()
                if h + 1 < CCW:
                    for r in ccw_pair(h + 1):
                        r.start()
            acc = combine(acc, cw_o[h], cw_s[h])
            if h < CCW:
                acc = combine(acc, ccw_o[h], ccw_s[h])

        acc_o, acc_m, acc_l = acc
        o = acc_o / _expand(acc_l)
        for b in range(B):
            out_ref[b] = jnp.dot(
                o[:, b * Dq : (b + 1) * Dq],
                wo_ref[...],
                preferred_element_type=jnp.float32,
            )

        for h in range(CW):
            for r in cw_pair(h):
                r.wait_send()
        for h in range(CCW):
            for r in ccw_pair(h):
                r.wait_send()

        @functools.partial(
            pl.run_scoped, second_barrier=pltpu.SemaphoreType.REGULAR
        )
        def _(second_barrier):
            for nbr in (left, right):
                pl.semaphore_signal(
                    second_barrier, inc=1,
                    device_id=(nbr,), device_id_type=pl.DeviceIdType.MESH,
                )
            pl.semaphore_wait(second_barrier, 2)

    return pl.pallas_call(
        body,
        out_shape=jax.ShapeDtypeStruct((B, Sq, D), jnp.float32),
        in_specs=[pl.BlockSpec(memory_space=pltpu.VMEM)] * 5,
        out_specs=pl.BlockSpec(memory_space=pltpu.VMEM),
        scratch_shapes=[
            pltpu.VMEM((Sq, W_O), jnp.float32),
            pltpu.VMEM((Sq, W_S), jnp.float32),
            pltpu.VMEM((CW, Sq, W_O), jnp.float32),
            pltpu.VMEM((CW, Sq, W_S), jnp.float32),
            pltpu.VMEM((CCW, Sq, W_O), jnp.float32),
            pltpu.VMEM((CCW, Sq, W_S), jnp.float32),
            pltpu.SemaphoreType.DMA((CW,)),
            pltpu.SemaphoreType.DMA((CW,)),
            pltpu.SemaphoreType.DMA((CW,)),
            pltpu.SemaphoreType.DMA((CW,)),
            pltpu.SemaphoreType.DMA((CCW,)),
            pltpu.SemaphoreType.DMA((CCW,)),
            pltpu.SemaphoreType.DMA((CCW,)),
            pltpu.SemaphoreType.DMA((CCW,)),
        ],
        compiler_params=pltpu.CompilerParams(
            collective_id=0, vmem_limit_bytes=100 * 1024 * 1024
        ),
    )(x, Wq, Wo, K2, V2)


# device time: 65069 ns/iter; 8.4859x vs baseline; 6.5771x over previous
import functools

import jax
import jax.numpy as jnp
from jax import lax
from jax.experimental import pallas as pl
from jax.experimental.pallas import tpu as pltpu

N_DEV = 32
B, Sq, D = 2, 256, 768
Hq, Dh = 8, 64
Dq = Hq * Dh
NB = B * Hq
W_O = B * Dq
W_C = W_O + 128
R = Sq // N_DEV
SCALE = 0.125


def _expand(a, rows):
    return jnp.concatenate(
        [jnp.broadcast_to(a[:, k : k + 1], (rows, Dh)) for k in range(NB)], axis=1
    )


def kernel(x, Wq, Wo, K_ext, V_ext):
    skv = K_ext.shape[1]

    K2 = jnp.transpose(K_ext, (0, 2, 3, 1)).reshape(NB, Dh, skv)
    V2 = jnp.transpose(V_ext, (0, 2, 1, 3)).reshape(NB, skv, Dh)

    def body(
        x_ref,
        wq_ref,
        wo_ref,
        k_ref,
        v_ref,
        out_ref,
        chunks,
        rs_buf,
        fin,
        ag_buf,
        ss1, rs1,
        ss2, rs2,
    ):
        my = lax.axis_index("i")

        barrier_sem = pltpu.get_barrier_semaphore()
        for j in range(N_DEV):
            @pl.when(my != j)
            def _():
                pl.semaphore_signal(
                    barrier_sem, inc=1,
                    device_id=(j,), device_id_type=pl.DeviceIdType.MESH,
                )
        pl.semaphore_wait(barrier_sem, N_DEV - 1)

        o_bands, m_cols, l_cols = [], [], []
        for b in range(B):
            qb = jnp.dot(x_ref[b], wq_ref[...], preferred_element_type=jnp.float32)
            for h in range(Hq):
                bh = b * Hq + h
                qh = qb[:, h * Dh : (h + 1) * Dh]
                s = (
                    jnp.dot(qh, k_ref[bh], preferred_element_type=jnp.float32)
                    * SCALE
                )
                mh = jnp.max(s, axis=1, keepdims=True)
                p = jnp.exp(s - mh)
                lh = jnp.sum(p, axis=1, keepdims=True)
                oh = jnp.dot(p, v_ref[bh], preferred_element_type=jnp.float32)
                o_bands.append(oh)
                m_cols.append(mh)
                l_cols.append(lh)
        acc_o = jnp.concatenate(o_bands, axis=1)
        acc_m = jnp.concatenate(m_cols, axis=1)
        acc_l = jnp.concatenate(l_cols, axis=1)
        pad = jnp.zeros((Sq, 128 - 2 * NB), dtype=jnp.float32)
        chunks[...] = jnp.concatenate([acc_o, acc_m, acc_l, pad], axis=1)

        def p1(j):
            return pltpu.make_async_remote_copy(
                src_ref=chunks.at[pl.ds(R * j, R)],
                dst_ref=rs_buf.at[pl.ds(my * R, R)],
                send_sem=ss1.at[j],
                recv_sem=rs1.at[my],
                device_id=(j,),
                device_id_type=pl.DeviceIdType.MESH,
            )

        for j in range(N_DEV):
            @pl.when(my != j)
            def _():
                p1(j).start()
        rs_buf[pl.ds(my * R, R), :] = chunks[pl.ds(my * R, R), :]

        def p1_recv(s):
            return pltpu.make_async_remote_copy(
                src_ref=chunks.at[pl.ds(R * s, R)],
                dst_ref=rs_buf.at[pl.ds(R * s, R)],
                send_sem=ss1.at[s],
                recv_sem=rs1.at[s],
                device_id=(s,),
                device_id_type=pl.DeviceIdType.MESH,
            )

        for s in range(N_DEV):
            @pl.when(my != s)
            def _():
                p1_recv(s).wait_recv()

        o = rs_buf[pl.ds(0, R), 0:W_O]
        m = rs_buf[pl.ds(0, R), W_O : W_O + NB]
        l = rs_buf[pl.ds(0, R), W_O + NB : W_O + 2 * NB]
        for s in range(1, N_DEV):
            o_in = rs_buf[pl.ds(R * s, R), 0:W_O]
            m_in = rs_buf[pl.ds(R * s, R), W_O : W_O + NB]
            l_in = rs_buf[pl.ds(R * s, R), W_O + NB : W_O + 2 * NB]
            m_new = jnp.maximum(m, m_in)
            a_acc = jnp.exp(m - m_new)
            a_in = jnp.exp(m_in - m_new)
            l = a_acc * l + a_in * l_in
            o = _expand(a_acc, R) * o + _expand(a_in, R) * o_in
            m = m_new
        fin[...] = o / _expand(l, R)

        def p2(j):
            return pltpu.make_async_remote_copy(
                src_ref=fin,
                dst_ref=ag_buf.at[pl.ds(my * R, R)],
                send_sem=ss2.at[j],
                recv_sem=rs2.at[my],
                device_id=(j,),
                device_id_type=pl.DeviceIdType.MESH,
            )

        for j in range(N_DEV):
            @pl.when(my != j)
            def _():
                p2(j).start()
        ag_buf[pl.ds(my * R, R), :] = fin[...]

        def p2_recv(s):
            return pltpu.make_async_remote_copy(
                src_ref=fin,
                dst_ref=ag_buf.at[pl.ds(R * s, R)],
                send_sem=ss2.at[s],
                recv_sem=rs2.at[s],
                device_id=(s,),
                device_id_type=pl.DeviceIdType.MESH,
            )

        for s in range(N_DEV):
            @pl.when(my != s)
            def _():
                p2_recv(s).wait_recv()

        for b in range(B):
            out_ref[b] = jnp.dot(
                ag_buf[:, b * Dq : (b + 1) * Dq],
                wo_ref[...],
                preferred_element_type=jnp.float32,
            )

        for j in range(N_DEV):
            @pl.when(my != j)
            def _():
                p1(j).wait_send()
                p2(j).wait_send()

        @functools.partial(
            pl.run_scoped, second_barrier=pltpu.SemaphoreType.REGULAR
        )
        def _(second_barrier):
            for j in range(N_DEV):
                @pl.when(my != j)
                def _():
                    pl.semaphore_signal(
                        second_barrier, inc=1,
                        device_id=(j,), device_id_type=pl.DeviceIdType.MESH,
                    )
            pl.semaphore_wait(second_barrier, N_DEV - 1)

    return pl.pallas_call(
        body,
        out_shape=jax.ShapeDtypeStruct((B, Sq, D), jnp.float32),
        in_specs=[pl.BlockSpec(memory_space=pltpu.VMEM)] * 5,
        out_specs=pl.BlockSpec(memory_space=pltpu.VMEM),
        scratch_shapes=[
            pltpu.VMEM((Sq, W_C), jnp.float32),
            pltpu.VMEM((Sq, W_C), jnp.float32),
            pltpu.VMEM((R, W_O), jnp.float32),
            pltpu.VMEM((Sq, W_O), jnp.float32),
            pltpu.SemaphoreType.DMA((N_DEV,)),
            pltpu.SemaphoreType.DMA((N_DEV,)),
            pltpu.SemaphoreType.DMA((N_DEV,)),
            pltpu.SemaphoreType.DMA((N_DEV,)),
        ],
        compiler_params=pltpu.CompilerParams(
            collective_id=0, vmem_limit_bytes=100 * 1024 * 1024
        ),
    )(x, Wq, Wo, K2, V2)


# device time: 64224 ns/iter; 8.5975x vs baseline; 1.0132x over previous
import jax
import jax.numpy as jnp
from jax import lax
from jax.experimental import pallas as pl
from jax.experimental.pallas import tpu as pltpu

N_DEV = 32
B, Sq, D = 2, 256, 768
Hq, Dh = 8, 64
Dq = Hq * Dh
NB = B * Hq
W_O = B * Dq
W_C = W_O + 128
R = Sq // N_DEV
G = 4
GR = Sq // G
SCALE = 0.125
MM = jnp.bfloat16


def _expand(a, rows):
    return jnp.concatenate(
        [jnp.broadcast_to(a[:, k : k + 1], (rows, Dh)) for k in range(NB)], axis=1
    )


def kernel(x, Wq, Wo, K_ext, V_ext):
    skv = K_ext.shape[1]

    K2 = jnp.transpose(K_ext, (0, 2, 3, 1)).reshape(NB, Dh, skv).astype(MM)
    V2 = jnp.transpose(V_ext, (0, 2, 1, 3)).reshape(NB, skv, Dh).astype(MM)

    def body(
        x_ref,
        wq_ref,
        wo_ref,
        k_ref,
        v_ref,
        out_ref,
        chunks,
        rs_buf,
        fin,
        ag_buf,
        ss1, rs1,
        ss2, rs2,
    ):
        my = lax.axis_index("i")

        barrier_sem = pltpu.get_barrier_semaphore()
        for j in range(N_DEV):
            @pl.when(my != j)
            def _():
                pl.semaphore_signal(
                    barrier_sem, inc=1,
                    device_id=(j,), device_id_type=pl.DeviceIdType.MESH,
                )

        def p1(j):
            return pltpu.make_async_remote_copy(
                src_ref=chunks.at[pl.ds(R * j, R)],
                dst_ref=rs_buf.at[pl.ds(my * R, R)],
                send_sem=ss1.at[j],
                recv_sem=rs1.at[my],
                device_id=(j,),
                device_id_type=pl.DeviceIdType.MESH,
            )

        wq_b = wq_ref[...].astype(MM)
        for g in range(G):
            r0 = GR * g
            o_bands, m_cols, l_cols = [], [], []
            for b in range(B):
                xg = x_ref[b, r0 : r0 + GR, :].astype(MM)
                qg = jnp.dot(xg, wq_b, preferred_element_type=jnp.float32)
                for h in range(Hq):
                    bh = b * Hq + h
                    qh = qg[:, h * Dh : (h + 1) * Dh].astype(MM)
                    s = (
                        jnp.dot(qh, k_ref[bh], preferred_element_type=jnp.float32)
                        * SCALE
                    )
                    mh = jnp.max(s, axis=1, keepdims=True)
                    p = jnp.exp(s - mh)
                    lh = jnp.sum(p, axis=1, keepdims=True)
                    oh = jnp.dot(
                        p.astype(MM), v_ref[bh], preferred_element_type=jnp.float32
                    )
                    o_bands.append(oh)
                    m_cols.append(mh)
                    l_cols.append(lh)
            pad = jnp.zeros((GR, 128 - 2 * NB), dtype=jnp.float32)
            chunks[r0 : r0 + GR, :] = jnp.concatenate(
                [
                    jnp.concatenate(o_bands, axis=1),
                    jnp.concatenate(m_cols, axis=1),
                    jnp.concatenate(l_cols, axis=1),
                    pad,
                ],
                axis=1,
            )
            if g == 0:
                pl.semaphore_wait(barrier_sem, N_DEV - 1)
            for jj in range(GR // R):
                j = (GR // R) * g + jj
                @pl.when(my != j)
                def _():
                    p1(j).start()

        rs_buf[pl.ds(my * R, R), :] = chunks[pl.ds(my * R, R), :]

        def p1_recv(s):
            return pltpu.make_async_remote_copy(
                src_ref=chunks.at[pl.ds(R * s, R)],
                dst_ref=rs_buf.at[pl.ds(R * s, R)],
                send_sem=ss1.at[s],
                recv_sem=rs1.at[s],
                device_id=(s,),
                device_id_type=pl.DeviceIdType.MESH,
            )

        for s in range(N_DEV):
            @pl.when(my != s)
            def _():
                p1_recv(s).wait_recv()

        o = rs_buf[:, 0:W_O]
        m = rs_buf[:, W_O : W_O + NB]
        l = rs_buf[:, W_O + NB : W_O + 2 * NB]
        rows = Sq
        while rows > R:
            half = rows // 2
            m_new = jnp.maximum(m[:half], m[half:rows])
            a1 = jnp.exp(m[:half] - m_new)
            a2 = jnp.exp(m[half:rows] - m_new)
            l = a1 * l[:half] + a2 * l[half:rows]
            o = _expand(a1, half) * o[:half] + _expand(a2, half) * o[half:rows]
            m = m_new
            rows = half
        fin[...] = o / _expand(l, R)

        def p2(j):
            return pltpu.make_async_remote_copy(
                src_ref=fin,
                dst_ref=ag_buf.at[pl.ds(my * R, R)],
                send_sem=ss2.at[j],
                recv_sem=rs2.at[my],
                device_id=(j,),
                device_id_type=pl.DeviceIdType.MESH,
            )

        for j in range(N_DEV):
            @pl.when(my != j)
            def _():
                p2(j).start()
        ag_buf[pl.ds(my * R, R), :] = fin[...]

        def p2_recv(s):
            return pltpu.make_async_remote_copy(
                src_ref=fin,
                dst_ref=ag_buf.at[pl.ds(R * s, R)],
                send_sem=ss2.at[s],
                recv_sem=rs2.at[s],
                device_id=(s,),
                device_id_type=pl.DeviceIdType.MESH,
            )

        for s in range(N_DEV):
            @pl.when(my != s)
            def _():
                p2_recv(s).wait_recv()

        wo_b = wo_ref[...].astype(MM)
        for b in range(B):
            out_ref[b] = jnp.dot(
                ag_buf[:, b * Dq : (b + 1) * Dq].astype(MM),
                wo_b,
                preferred_element_type=jnp.float32,
            )

        for j in range(N_DEV):
            @pl.when(my != j)
            def _():
                p1(j).wait_send()
                p2(j).wait_send()

    return pl.pallas_call(
        body,
        out_shape=jax.ShapeDtypeStruct((B, Sq, D), jnp.float32),
        in_specs=[pl.BlockSpec(memory_space=pltpu.VMEM)] * 5,
        out_specs=pl.BlockSpec(memory_space=pltpu.VMEM),
        scratch_shapes=[
            pltpu.VMEM((Sq, W_C), jnp.float32),
            pltpu.VMEM((Sq, W_C), jnp.float32),
            pltpu.VMEM((R, W_O), jnp.float32),
            pltpu.VMEM((Sq, W_O), jnp.float32),
            pltpu.SemaphoreType.DMA((N_DEV,)),
            pltpu.SemaphoreType.DMA((N_DEV,)),
            pltpu.SemaphoreType.DMA((N_DEV,)),
            pltpu.SemaphoreType.DMA((N_DEV,)),
        ],
        compiler_params=pltpu.CompilerParams(
            collective_id=0, vmem_limit_bytes=100 * 1024 * 1024
        ),
    )(x, Wq, Wo, K2, V2)


# device time: 53577 ns/iter; 10.3061x vs baseline; 1.1987x over previous
import jax
import jax.numpy as jnp
from jax import lax
from jax.experimental import pallas as pl
from jax.experimental.pallas import tpu as pltpu

N_DEV = 32
B, Sq, D = 2, 256, 768
Hq, Dh = 8, 64
Dq = Hq * Dh
NB = B * Hq
W_O = B * Dq
R = Sq // N_DEV
G = 4
GR = Sq // G
OPG = GR // R
SCALE = 0.125
MM = jnp.bfloat16


def _expand(a, rows):
    return jnp.concatenate(
        [jnp.broadcast_to(a[:, k : k + 1], (rows, Dh)) for k in range(NB)], axis=1
    )


def kernel(x, Wq, Wo, K_ext, V_ext):
    skv = K_ext.shape[1]

    K2 = jnp.transpose(K_ext, (0, 2, 3, 1)).reshape(NB, Dh, skv).astype(MM)
    V2 = jnp.transpose(V_ext, (0, 2, 1, 3)).reshape(NB, skv, Dh).astype(MM)

    def body(
        x_ref,
        wq_ref,
        wo_ref,
        k_ref,
        v_ref,
        out_ref,
        chunks_o,
        chunks_s,
        rs_o,
        rs_s,
        fin,
        ag_buf,
        ss1o, rs1o, ss1s, rs1s,
        ss2, rs2,
    ):
        my = lax.axis_index("i")

        barrier_sem = pltpu.get_barrier_semaphore()
        for j in range(N_DEV):
            @pl.when(my != j)
            def _():
                pl.semaphore_signal(
                    barrier_sem, inc=1,
                    device_id=(j,), device_id_type=pl.DeviceIdType.MESH,
                )

        def p1(j, send):
            dst = my if send else j
            ro = pltpu.make_async_remote_copy(
                src_ref=chunks_o.at[:, pl.ds(512 * j, 512)],
                dst_ref=rs_o.at[:, pl.ds(dst * 512, 512)],
                send_sem=ss1o.at[j],
                recv_sem=rs1o.at[dst],
                device_id=(j,),
                device_id_type=pl.DeviceIdType.MESH,
            )
            rs_ = pltpu.make_async_remote_copy(
                src_ref=chunks_s.at[:, pl.ds(128 * j, 128)],
                dst_ref=rs_s.at[:, pl.ds(dst * 128, 128)],
                send_sem=ss1s.at[j],
                recv_sem=rs1s.at[dst],
                device_id=(j,),
                device_id_type=pl.DeviceIdType.MESH,
            )
            return ro, rs_

        wq_b = wq_ref[...].astype(MM)
        for g in range(G):
            r0 = GR * g
            o_bands, m_cols, l_cols = [], [], []
            for b in range(B):
                xg = x_ref[b, r0 : r0 + GR, :].astype(MM)
                qg = jnp.dot(xg, wq_b, preferred_element_type=jnp.float32)
                for h in range(Hq):
                    bh = b * Hq + h
                    qh = qg[:, h * Dh : (h + 1) * Dh].astype(MM)
                    s = (
                        jnp.dot(qh, k_ref[bh], preferred_element_type=jnp.float32)
                        * SCALE
                    )
                    mh = jnp.max(s, axis=1, keepdims=True)
                    p = jnp.exp(s - mh)
                    lh = jnp.sum(p, axis=1, keepdims=True)
                    oh = jnp.dot(
                        p.astype(MM), v_ref[bh], preferred_element_type=jnp.float32
                    )
                    o_bands.append(oh)
                    m_cols.append(mh)
                    l_cols.append(lh)
            o_g = jnp.concatenate(o_bands, axis=1)
            m_g = jnp.concatenate(m_cols, axis=1)
            l_g = jnp.concatenate(l_cols, axis=1)
            spad = jnp.zeros((R, 128 - 2 * NB), dtype=jnp.float32)
            for jj in range(OPG):
                j = OPG * g + jj
                rsl = slice(R * jj, R * (jj + 1))
                ob = o_g[rsl]
                chunks_o[:, 512 * j : 512 * (j + 1)] = jnp.concatenate(
                    [ob[:, 0:Dq], ob[:, Dq:W_O]], axis=0
                ).astype(MM)
                chunks_s[:, 128 * j : 128 * (j + 1)] = jnp.concatenate(
                    [m_g[rsl], l_g[rsl], spad], axis=1
                )
            if g == 0:
                pl.semaphore_wait(barrier_sem, N_DEV - 1)
            for jj in range(OPG):
                j = OPG * g + jj
                @pl.when(my != j)
                def _():
                    for r in p1(j, send=True):
                        r.start()

        rs_o[:, pl.ds(my * 512, 512)] = chunks_o[:, pl.ds(my * 512, 512)]
        rs_s[:, pl.ds(my * 128, 128)] = chunks_s[:, pl.ds(my * 128, 128)]

        for s in range(N_DEV):
            @pl.when(my != s)
            def _():
                for r in p1(s, send=False):
                    r.wait_recv()

        o_parts, m_parts, l_parts = [], [], []
        for s in range(N_DEV):
            blk = rs_o[:, 512 * s : 512 * (s + 1)]
            o_parts.append(
                jnp.concatenate([blk[0:R], blk[R : 2 * R]], axis=1).astype(
                    jnp.float32
                )
            )
            sblk = rs_s[:, 128 * s : 128 * (s + 1)]
            m_parts.append(sblk[:, 0:NB])
            l_parts.append(sblk[:, NB : 2 * NB])
        o = jnp.concatenate(o_parts, axis=0)
        m = jnp.concatenate(m_parts, axis=0)
        l = jnp.concatenate(l_parts, axis=0)
        rows = Sq
        while rows > R:
            half = rows // 2
            m_new = jnp.maximum(m[:half], m[half:rows])
            a1 = jnp.exp(m[:half] - m_new)
            a2 = jnp.exp(m[half:rows] - m_new)
            l = a1 * l[:half] + a2 * l[half:rows]
            o = _expand(a1, half) * o[:half] + _expand(a2, half) * o[half:rows]
            m = m_new
            rows = half
        o_fin = o / _expand(l, R)
        fin[...] = jnp.concatenate(
            [o_fin[:, 0:Dq], o_fin[:, Dq:W_O]], axis=0
        ).astype(MM)

        def p2(j, send):
            dst = my if send else j
            return pltpu.make_async_remote_copy(
                src_ref=fin,
                dst_ref=ag_buf.at[:, pl.ds(dst * 512, 512)],
                send_sem=ss2.at[j],
                recv_sem=rs2.at[dst],
                device_id=(j,),
                device_id_type=pl.DeviceIdType.MESH,
            )

        for j in range(N_DEV):
            @pl.when(my != j)
            def _():
                p2(j, send=True).start()
        ag_buf[:, pl.ds(my * 512, 512)] = fin[...]

        for s in range(N_DEV):
            @pl.when(my != s)
            def _():
                p2(s, send=False).wait_recv()

        o_rows = []
        for j in range(N_DEV):
            blk = ag_buf[:, 512 * j : 512 * (j + 1)]
            o_rows.append(jnp.concatenate([blk[0:R], blk[R : 2 * R]], axis=1))
        o_full = jnp.concatenate(o_rows, axis=0)
        wo_b = wo_ref[...].astype(MM)
        for b in range(B):
            out_ref[b] = jnp.dot(
                o_full[:, b * Dq : (b + 1) * Dq],
                wo_b,
                preferred_element_type=jnp.float32,
            )

        for j in range(N_DEV):
            @pl.when(my != j)
            def _():
                for r in p1(j, send=True):
                    r.wait_send()
                p2(j, send=True).wait_send()

    return pl.pallas_call(
        body,
        out_shape=jax.ShapeDtypeStruct((B, Sq, D), jnp.float32),
        in_specs=[pl.BlockSpec(memory_space=pltpu.VMEM)] * 5,
        out_specs=pl.BlockSpec(memory_space=pltpu.VMEM),
        scratch_shapes=[
            pltpu.VMEM((2 * R, N_DEV * Dq), MM),
            pltpu.VMEM((R, N_DEV * 128), jnp.float32),
            pltpu.VMEM((2 * R, N_DEV * Dq), MM),
            pltpu.VMEM((R, N_DEV * 128), jnp.float32),
            pltpu.VMEM((2 * R, Dq), MM),
            pltpu.VMEM((2 * R, N_DEV * Dq), MM),
            pltpu.SemaphoreType.DMA((N_DEV,)),
            pltpu.SemaphoreType.DMA((N_DEV,)),
            pltpu.SemaphoreType.DMA((N_DEV,)),
            pltpu.SemaphoreType.DMA((N_DEV,)),
            pltpu.SemaphoreType.DMA((N_DEV,)),
            pltpu.SemaphoreType.DMA((N_DEV,)),
        ],
        compiler_params=pltpu.CompilerParams(
            collective_id=0, vmem_limit_bytes=100 * 1024 * 1024
        ),
    )(x, Wq, Wo, K2, V2)
